# baseline (speedup 1.0000x reference)
import jax
import jax.numpy as jnp
from jax.experimental import pallas as pl
from jax.experimental.pallas import tpu as pltpu

_NBUF = 8
_CH = 512


def _router_body(
    x_hbm, m_hbm, w_ref, probs_hbm, logits_hbm,
    xbuf, mbuf, pbuf, lbuf, xsem, msem, psem, lsem,
):
    T = x_hbm.shape[0]
    nch = T // _CH
    w = w_ref[...]

    def x_copy(c, b):
        return pltpu.make_async_copy(
            x_hbm.at[pl.ds(c * _CH, _CH), :], xbuf.at[b], xsem.at[b]
        )

    def m_copy(c, b):
        return pltpu.make_async_copy(
            m_hbm.at[pl.ds(c * _CH, _CH), :], mbuf.at[b], msem.at[b]
        )

    def p_copy(c, b):
        return pltpu.make_async_copy(
            pbuf.at[b], probs_hbm.at[pl.ds(c * _CH, _CH), :], psem.at[b]
        )

    def l_copy(c, b):
        return pltpu.make_async_copy(
            lbuf.at[b], logits_hbm.at[pl.ds(c * _CH, _CH), :], lsem.at[b]
        )

    for i in range(_NBUF):
        x_copy(i, i).start()
        m_copy(i, i).start()

    for c in range(nch):
        b = c % _NBUF
        x_copy(c, b).wait()
        m_copy(c, b).wait()
        if c >= _NBUF:
            p_copy(c - _NBUF, b).wait()
            l_copy(c - _NBUF, b).wait()
        x = xbuf[b]
        logits = jnp.dot(x, w, preferred_element_type=jnp.float32)
        mx = jnp.max(logits, axis=-1, keepdims=True)
        e = jnp.exp(logits - mx)
        s = jnp.sum(e, axis=-1, keepdims=True)
        pbuf[b] = (e / s) * mbuf[b]
        lbuf[b] = logits
        p_copy(c, b).start()
        l_copy(c, b).start()
        nxt = c + _NBUF
        if nxt < nch:
            x_copy(nxt, b).start()
            m_copy(nxt, b).start()

    for i in range(_NBUF):
        c = nch - _NBUF + i
        p_copy(c, c % _NBUF).wait()
        l_copy(c, c % _NBUF).wait()


def kernel(inputs, padding_mask, w, num_experts):
    T, D = inputs.shape
    E = w.shape[1]
    probs, logits = pl.pallas_call(
        _router_body,
        in_specs=[
            pl.BlockSpec(memory_space=pl.ANY),
            pl.BlockSpec(memory_space=pl.ANY),
            pl.BlockSpec(memory_space=pltpu.VMEM),
        ],
        out_specs=[
            pl.BlockSpec(memory_space=pl.ANY),
            pl.BlockSpec(memory_space=pl.ANY),
        ],
        out_shape=[
            jax.ShapeDtypeStruct((T, E), jnp.float32),
            jax.ShapeDtypeStruct((T, E), jnp.float32),
        ],
        scratch_shapes=[
            pltpu.VMEM((_NBUF, _CH, D), jnp.float32),
            pltpu.VMEM((_NBUF, _CH, 1), jnp.float32),
            pltpu.VMEM((_NBUF, _CH, E), jnp.float32),
            pltpu.VMEM((_NBUF, _CH, E), jnp.float32),
            pltpu.SemaphoreType.DMA((_NBUF,)),
            pltpu.SemaphoreType.DMA((_NBUF,)),
            pltpu.SemaphoreType.DMA((_NBUF,)),
            pltpu.SemaphoreType.DMA((_NBUF,)),
        ],
    )(inputs, padding_mask, w)
    return (probs, logits)
